# Initial kernel scaffold; baseline (speedup 1.0000x reference)
import jax
import jax.numpy as jnp
from jax.experimental import pallas as pl


def kernel(features, edge_index, W1, b1, W2, b2):
    raise NotImplementedError("write your pallas kernel here")


# speedup vs baseline: 4.4421x; 4.4421x over previous
import functools

import jax
import jax.numpy as jnp
from jax import lax
from jax.experimental import pallas as pl
from jax.experimental.pallas import tpu as pltpu
from jax.experimental.pallas import tpu_sc as plsc

N = 10000
E = 320000
D = 128
DP = 144
NR = 10016
NC = 2
NS = 16
NW = NC * NS
BLK = 128
BPW = 79
E_PAD = NW * BPW * BLK
RPT = N // NS

_mesh = plsc.VectorSubcoreMesh(core_axis_name="c", subcore_axis_name="s")


@functools.partial(
    pl.kernel,
    out_type=jax.ShapeDtypeStruct((NC, N, DP), jnp.float32),
    mesh=_mesh,
    scratch_types=[
        pltpu.VMEM((BPW, BLK), jnp.int32),
        pltpu.VMEM((BPW, BLK), jnp.int32),
        pltpu.VMEM((BLK, DP), jnp.float32),
        pltpu.VMEM_SHARED((N, DP), jnp.float32),
        pltpu.SemaphoreType.DMA,
    ],
    compiler_params=pltpu.CompilerParams(use_tc_tiling_on_sc=False),
)
def _sc_aggregate(feat_hbm, srcb_hbm, dstb_hbm, out_hbm,
                  src_v, dst_v, rows_v, acc, sem):
    cid = lax.axis_index("c")
    sid = lax.axis_index("s")
    wid = sid * NC + cid

    zero = jnp.zeros((16,), jnp.float32)

    @pl.loop(0, BLK)
    def _(r):
        for c in range(DP // 16):
            rows_v[r, pl.ds(c * 16, 16)] = zero

    base = sid * RPT

    @pl.loop(0, RPT // BLK)
    def _(z):
        pltpu.sync_copy(rows_v, acc.at[pl.ds(base + z * BLK, BLK)])

    pltpu.sync_copy(rows_v.at[pl.ds(0, RPT % BLK)],
                    acc.at[pl.ds(base + RPT - RPT % BLK, RPT % BLK)])

    pltpu.sync_copy(srcb_hbm.at[wid], src_v)
    pltpu.sync_copy(dstb_hbm.at[wid], dst_v)

    plsc.subcore_barrier()

    @pl.loop(0, BPW)
    def _(j):
        pltpu.async_copy(feat_hbm.at[src_v.at[j]], rows_v, sem).wait()
        pltpu.sync_copy(rows_v, acc.at[dst_v.at[j]], add=True)

    plsc.subcore_barrier()

    pltpu.sync_copy(acc.at[pl.ds(base, RPT)],
                    out_hbm.at[cid, pl.ds(base, RPT)])


ROW_BLK = 400


def _mlp_body(x_ref, agg_ref, wt_ref, b_ref, out_ref, *, relu, pad_out):
    a = agg_ref[0] + agg_ref[1]
    deg = a[:, D:D + 1]
    mean = a[:, :D] / jnp.maximum(deg, 1.0)
    rst = x_ref[:, :D] + mean
    y = jnp.dot(rst, wt_ref[...], preferred_element_type=jnp.float32) + b_ref[...]
    if relu:
        y = jnp.maximum(y, 0.0)
    if pad_out:
        out_ref[:, :D] = y
        out_ref[:, D:D + 1] = jnp.ones((ROW_BLK, 1), jnp.float32)
        out_ref[:, D + 1:] = jnp.zeros((ROW_BLK, DP - D - 1), jnp.float32)
    else:
        out_ref[...] = y


def _mlp(x, agg, wt, b, *, relu, pad_out):
    din = x.shape[1]
    dout = DP if pad_out else D
    return pl.pallas_call(
        functools.partial(_mlp_body, relu=relu, pad_out=pad_out),
        grid=(N // ROW_BLK,),
        in_specs=[
            pl.BlockSpec((ROW_BLK, din), lambda i: (i, 0)),
            pl.BlockSpec((NC, ROW_BLK, DP), lambda i: (0, i, 0)),
            pl.BlockSpec((D, D), lambda i: (0, 0)),
            pl.BlockSpec((1, D), lambda i: (0, 0)),
        ],
        out_specs=pl.BlockSpec((ROW_BLK, dout), lambda i: (i, 0)),
        out_shape=jax.ShapeDtypeStruct((N, dout), jnp.float32),
    )(x, agg, wt, b)


@jax.jit
def _run(features, edge_index, W1, b1, W2, b2):
    src = jnp.pad(edge_index[0], (0, E_PAD - E),
                  constant_values=N).reshape(NW, BPW, BLK)
    dst = jnp.pad(edge_index[1], (0, E_PAD - E)).reshape(NW, BPW, BLK)
    x_aug = jnp.concatenate(
        [features,
         jnp.ones((N, 1), jnp.float32),
         jnp.zeros((N, DP - D - 1), jnp.float32)], axis=1)
    x_aug_p = jnp.pad(x_aug, ((0, NR - N), (0, 0)))

    agg1 = _sc_aggregate(x_aug_p, src, dst)
    x1_aug = _mlp(features, agg1, W1.T, b1[None, :], relu=True, pad_out=True)
    agg2 = _sc_aggregate(jnp.pad(x1_aug, ((0, NR - N), (0, 0))), src, dst)
    out = _mlp(x1_aug, agg2, W2.T, b2[None, :], relu=False, pad_out=False)
    return out


def kernel(features, edge_index, W1, b1, W2, b2):
    return _run(features, edge_index, W1, b1, W2, b2)
